# baseline (speedup 1.0000x reference)
import functools

import jax
import jax.numpy as jnp
import numpy as np
from jax import lax
from jax.experimental import pallas as pl
from jax.experimental.pallas import tpu as pltpu
from jax.experimental.pallas import tpu_sc as plsc

B = 2
C_IN = 96
H = 224
W = 224
SCALE = 2
KTAP = 4
SAMPLES = 16
C1 = (C_IN + 2) * SCALE * SCALE
H1 = H // SCALE
W1 = W // SCALE
N = H1 * W1
M = SAMPLES * SAMPLES
BN = B * N
CO = 96 * SCALE * SCALE
C98 = C_IN + 2
CP = 512
CF = KTAP * CP

TBLK = 896
NBLK = N // TBLK

NSPLIT = 2
NH = N // NSPLIT
BNH = B * NH
NBLKH = NH // TBLK

NC = 2
NS = 16
NW = NC * NS
UCH = 128
NUNIT = BNH // UCH

_x_ind = np.round(np.linspace(0, H1 - 1, SAMPLES)).astype(np.int32)
_y_ind = np.round(np.linspace(0, W1 - 1, SAMPLES)).astype(np.int32)
_xg, _yg = np.meshgrid(_x_ind, _y_ind, indexing="ij")
_FLAT_IDX = (_xg.flatten() * W1 + _yg.flatten()).astype(np.int32)
_PIN = np.full((N,), -1, dtype=np.int32)
_PIN[_FLAT_IDX] = np.arange(M, dtype=np.int32)


def _coords():
    xg, yg = jnp.meshgrid(
        jnp.arange(H, dtype=jnp.float32),
        jnp.arange(W, dtype=jnp.float32),
        indexing="ij",
    )
    xy = jnp.stack([xg, yg], axis=0)
    nrm = jnp.sqrt(jnp.sum(xy * xy, axis=0, keepdims=True))
    return xy / jnp.maximum(nrm, 1e-12)


def _stage1_body(xs_ref, wk_ref, bk_ref, wv_ref, bv_ref, knt_ref, vtab_ref):
    xs = xs_ref[0]
    k = jnp.dot(wk_ref[...], xs, preferred_element_type=jnp.float32) + bk_ref[...]
    knrm = jnp.sqrt(jnp.sum(k * k, axis=0, keepdims=True))
    knt_ref[0] = k / jnp.maximum(knrm, 1e-12)
    v = jnp.dot(wv_ref[...], xs, preferred_element_type=jnp.float32) + bv_ref[...]
    vtab_ref[...] = jnp.pad(v.T, ((0, 0), (0, CP - C1)))


def _stage1(xs, wk, bk2, wv, bv2):
    return pl.pallas_call(
        _stage1_body,
        grid=(B,),
        in_specs=[
            pl.BlockSpec((1, C1, M), lambda b: (b, 0, 0)),
            pl.BlockSpec((C1, C1), lambda b: (0, 0)),
            pl.BlockSpec((C1, 1), lambda b: (0, 0)),
            pl.BlockSpec((C1, C1), lambda b: (0, 0)),
            pl.BlockSpec((C1, 1), lambda b: (0, 0)),
        ],
        out_specs=[
            pl.BlockSpec((1, C1, M), lambda b: (b, 0, 0)),
            pl.BlockSpec((M, CP), lambda b: (b, 0)),
        ],
        out_shape=[
            jax.ShapeDtypeStruct((B, C1, M), jnp.float32),
            jax.ShapeDtypeStruct((B * M, CP), jnp.float32),
        ],
    )(xs, wk, bk2, wv, bv2)


def _stage2_body(x_ref, wq_ref, bq_ref, knt_ref, pin_ref,
                 g0_ref, g1_ref, g2_ref, g3_ref, attn_ref):
    b = pl.program_id(0)
    x = x_ref[0]
    q = jnp.dot(wq_ref[...], x, preferred_element_type=jnp.float32) + bq_ref[...]
    qnrm = jnp.sqrt(jnp.sum(q * q, axis=0, keepdims=True))
    qn = q / jnp.maximum(qnrm, 1e-12)
    sim = lax.dot_general(
        qn, knt_ref[0], (((0,), (0,)), ((), ())),
        preferred_element_type=jnp.float32,
    )
    col = lax.broadcasted_iota(jnp.int32, (TBLK, M), 1)
    pin = pin_ref[0, 0]
    sim = jnp.where(col == pin[:, None], 1e9, sim)
    vals = []
    idxs = []
    for kt in range(KTAP):
        mval = jnp.max(sim, axis=1)
        midx = jnp.min(jnp.where(sim == mval[:, None], col, M), axis=1)
        vals.append(mval)
        idxs.append(midx)
        if kt + 1 < KTAP:
            sim = jnp.where(col == midx[:, None], -jnp.inf, sim)
    vv = jnp.stack(vals, axis=0)
    e = jnp.exp(vv - vv[0][None, :])
    attn_ref[...] = e / jnp.sum(e, axis=0, keepdims=True)
    for kt, gref in enumerate((g0_ref, g1_ref, g2_ref, g3_ref)):
        gref[0] = idxs[kt] + b * M


def _stage2(x2, wq, bq2, knt, pin3, h):
    return pl.pallas_call(
        _stage2_body,
        grid=(B, NBLKH),
        in_specs=[
            pl.BlockSpec((1, C1, TBLK), lambda b, nb: (b, 0, h * NBLKH + nb)),
            pl.BlockSpec((C1, C1), lambda b, nb: (0, 0)),
            pl.BlockSpec((C1, 1), lambda b, nb: (0, 0)),
            pl.BlockSpec((1, C1, M), lambda b, nb: (b, 0, 0)),
            pl.BlockSpec((1, 1, TBLK), lambda b, nb: (h * NBLKH + nb, 0, 0)),
        ],
        out_specs=[
            pl.BlockSpec((1, TBLK), lambda b, nb: (0, b * NBLKH + nb)),
            pl.BlockSpec((1, TBLK), lambda b, nb: (0, b * NBLKH + nb)),
            pl.BlockSpec((1, TBLK), lambda b, nb: (0, b * NBLKH + nb)),
            pl.BlockSpec((1, TBLK), lambda b, nb: (0, b * NBLKH + nb)),
            pl.BlockSpec((KTAP, TBLK), lambda b, nb: (0, b * NBLKH + nb)),
        ],
        out_shape=[
            jax.ShapeDtypeStruct((1, BNH), jnp.int32),
            jax.ShapeDtypeStruct((1, BNH), jnp.int32),
            jax.ShapeDtypeStruct((1, BNH), jnp.int32),
            jax.ShapeDtypeStruct((1, BNH), jnp.int32),
            jax.ShapeDtypeStruct((KTAP, BNH), jnp.float32),
        ],
    )(x2, wq, bq2, knt, pin3)


def _stage3_body(vtab_hbm, g0_hbm, g1_hbm, g2_hbm, g3_hbm, g_hbm,
                 idx_v, rows_v, sem):
    wid = lax.axis_index("s") * NC + lax.axis_index("c")
    gidx_hbms = (g0_hbm, g1_hbm, g2_hbm, g3_hbm)

    for j in range((NUNIT + NW - 1) // NW):
        u = wid + j * NW

        @pl.when(u < NUNIT)
        def _():
            off = u * UCH
            for kt in range(KTAP):
                pltpu.sync_copy(gidx_hbms[kt].at[0, pl.ds(off, UCH)], idx_v[kt])
            for kt in range(KTAP):
                pltpu.async_copy(vtab_hbm.at[idx_v[kt]], rows_v, sem).wait()
                pltpu.sync_copy(rows_v, g_hbm.at[kt, pl.ds(off, UCH), :])


@functools.lru_cache(maxsize=1)
def _make_stage3():
    return pl.kernel(
        _stage3_body,
        out_type=jax.ShapeDtypeStruct((KTAP, BNH, CP), jnp.float32),
        mesh=plsc.VectorSubcoreMesh(core_axis_name="c", subcore_axis_name="s"),
        compiler_params=pltpu.CompilerParams(use_tc_tiling_on_sc=True),
        scratch_types=[
            [pltpu.VMEM((UCH,), jnp.int32) for _ in range(KTAP)],
            pltpu.VMEM((UCH, CP), jnp.float32),
            pltpu.SemaphoreType.DMA,
        ],
    )


def _stage4_body(g_ref, attn_ref, wflat_ref, cb_ref, pfold_ref, pb_ref, out_ref):
    attn = attn_ref[...]
    parts = [g_ref[kt] * attn[kt][:, None] for kt in range(KTAP)]
    weighted = jnp.concatenate(parts, axis=1)
    out1d = (
        jnp.dot(weighted, wflat_ref[...], preferred_element_type=jnp.float32)
        + cb_ref[...]
    )
    out_ref[...] = (
        jnp.dot(out1d, pfold_ref[...], preferred_element_type=jnp.float32)
        + pb_ref[...]
    )


def _stage4(g, attn, wflat, cb2, pfold, pb2):
    nblk2 = BNH // TBLK
    return pl.pallas_call(
        _stage4_body,
        grid=(nblk2,),
        in_specs=[
            pl.BlockSpec((KTAP, TBLK, CP), lambda i: (0, i, 0)),
            pl.BlockSpec((KTAP, TBLK), lambda i: (0, i)),
            pl.BlockSpec((CF, C1), lambda i: (0, 0)),
            pl.BlockSpec((1, C1), lambda i: (0, 0)),
            pl.BlockSpec((C1, CO), lambda i: (0, 0)),
            pl.BlockSpec((1, CO), lambda i: (0, 0)),
        ],
        out_specs=pl.BlockSpec((TBLK, CO), lambda i: (i, 0)),
        out_shape=jax.ShapeDtypeStruct((BNH, CO), jnp.float32),
    )(g, attn, wflat, cb2, pfold, pb2)


def kernel(x, Wq, bq, Wk, bk, Wv, bv, conv_w, conv_b, pw_w, pw_b):
    coords = jnp.broadcast_to(_coords()[None], (B, 2, H, W))
    xc = jnp.concatenate([x, coords], axis=1)
    x2 = (
        xc.reshape(B, C98, H1, SCALE, W1, SCALE)
        .transpose(0, 1, 3, 5, 2, 4)
        .reshape(B, C1, N)
    )
    xs = x2[:, :, jnp.asarray(_FLAT_IDX)]

    wflat = jnp.pad(
        conv_w.transpose(2, 1, 0), ((0, 0), (0, CP - C1), (0, 0))
    ).reshape(CF, C1)
    eye4 = jnp.eye(SCALE * SCALE, dtype=pw_w.dtype)
    pfold = jnp.einsum(
        "oc,st->ctso", pw_w, eye4, precision=jax.lax.Precision.HIGHEST
    ).reshape(C1, CO)
    bq2 = bq.reshape(C1, 1)
    bk2 = bk.reshape(C1, 1)
    bv2 = bv.reshape(C1, 1)
    cb2 = conv_b.reshape(1, C1)
    pb2 = jnp.tile(pw_b, SCALE * SCALE).reshape(1, CO)
    pin3 = jnp.asarray(_PIN).reshape(NBLK, 1, TBLK)

    knt, vflat = _stage1(xs, Wk, bk2, Wv, bv2)

    outs = []
    for h in range(NSPLIT):
        gi0, gi1, gi2, gi3, attn = _stage2(x2, Wq, bq2, knt, pin3, h)
        g = _make_stage3()(vflat, gi0, gi1, gi2, gi3)
        outs.append(_stage4(g, attn, wflat, cb2, pfold, pb2))

    out_tok = jnp.concatenate(
        [o.reshape(B, NH, CO) for o in outs], axis=1
    )

    x5 = (
        out_tok.reshape(B, H1, W1, SCALE, SCALE, C_IN)
        .transpose(0, 5, 1, 3, 2, 4)
        .reshape(B, C_IN, H, W)
    )
    return x5


# speedup vs baseline: 1.0031x; 1.0031x over previous
import functools

import jax
import jax.numpy as jnp
import numpy as np
from jax import lax
from jax.experimental import pallas as pl
from jax.experimental.pallas import tpu as pltpu
from jax.experimental.pallas import tpu_sc as plsc

B = 2
C_IN = 96
H = 224
W = 224
SCALE = 2
KTAP = 4
SAMPLES = 16
C1 = (C_IN + 2) * SCALE * SCALE
H1 = H // SCALE
W1 = W // SCALE
N = H1 * W1
M = SAMPLES * SAMPLES
BN = B * N
CO = 96 * SCALE * SCALE
C98 = C_IN + 2
CP = 512
CF = KTAP * CP

TBLK = 896
NBLK = N // TBLK

NSPLIT = 2
NH = N // NSPLIT
BNH = B * NH
NBLKH = NH // TBLK

NC = 2
NS = 16
NW = NC * NS
UCH = 128
NUNIT = BNH // UCH

_x_ind = np.round(np.linspace(0, H1 - 1, SAMPLES)).astype(np.int32)
_y_ind = np.round(np.linspace(0, W1 - 1, SAMPLES)).astype(np.int32)
_xg, _yg = np.meshgrid(_x_ind, _y_ind, indexing="ij")
_FLAT_IDX = (_xg.flatten() * W1 + _yg.flatten()).astype(np.int32)
_PIN = np.full((N,), -1, dtype=np.int32)
_PIN[_FLAT_IDX] = np.arange(M, dtype=np.int32)


def _coords():
    xg, yg = jnp.meshgrid(
        jnp.arange(H, dtype=jnp.float32),
        jnp.arange(W, dtype=jnp.float32),
        indexing="ij",
    )
    xy = jnp.stack([xg, yg], axis=0)
    nrm = jnp.sqrt(jnp.sum(xy * xy, axis=0, keepdims=True))
    return xy / jnp.maximum(nrm, 1e-12)


def _stage1_body(xs_ref, wk_ref, bk_ref, wv_ref, bv_ref, knt_ref, vtab_ref):
    xs = xs_ref[0]
    k = jnp.dot(wk_ref[...], xs, preferred_element_type=jnp.float32) + bk_ref[...]
    knrm = jnp.sqrt(jnp.sum(k * k, axis=0, keepdims=True))
    knt_ref[0] = k / jnp.maximum(knrm, 1e-12)
    v = jnp.dot(wv_ref[...], xs, preferred_element_type=jnp.float32) + bv_ref[...]
    vtab_ref[...] = jnp.pad(v.T, ((0, 0), (0, CP - C1)))


def _stage1(xs, wk, bk2, wv, bv2):
    return pl.pallas_call(
        _stage1_body,
        grid=(B,),
        in_specs=[
            pl.BlockSpec((1, C1, M), lambda b: (b, 0, 0)),
            pl.BlockSpec((C1, C1), lambda b: (0, 0)),
            pl.BlockSpec((C1, 1), lambda b: (0, 0)),
            pl.BlockSpec((C1, C1), lambda b: (0, 0)),
            pl.BlockSpec((C1, 1), lambda b: (0, 0)),
        ],
        out_specs=[
            pl.BlockSpec((1, C1, M), lambda b: (b, 0, 0)),
            pl.BlockSpec((M, CP), lambda b: (b, 0)),
        ],
        out_shape=[
            jax.ShapeDtypeStruct((B, C1, M), jnp.float32),
            jax.ShapeDtypeStruct((B * M, CP), jnp.float32),
        ],
    )(xs, wk, bk2, wv, bv2)


def _stage2_body(x_ref, wq_ref, bq_ref, knt_ref, pin_ref,
                 g0_ref, g1_ref, g2_ref, g3_ref, attn_ref):
    b = pl.program_id(0)
    x = x_ref[0]
    q = jnp.dot(wq_ref[...], x, preferred_element_type=jnp.float32) + bq_ref[...]
    qnrm = jnp.sqrt(jnp.sum(q * q, axis=0, keepdims=True))
    qn = q / jnp.maximum(qnrm, 1e-12)
    sim = lax.dot_general(
        qn, knt_ref[0], (((0,), (0,)), ((), ())),
        preferred_element_type=jnp.float32,
    )
    col = lax.broadcasted_iota(jnp.int32, (TBLK, M), 1)
    pin = pin_ref[0, 0]
    sim = jnp.where(col == pin[:, None], 1e9, sim)
    vals = []
    idxs = []
    for kt in range(KTAP):
        mval = jnp.max(sim, axis=1)
        midx = jnp.min(jnp.where(sim == mval[:, None], col, M), axis=1)
        vals.append(mval)
        idxs.append(midx)
        if kt + 1 < KTAP:
            sim = jnp.where(col == midx[:, None], -jnp.inf, sim)
    vv = jnp.stack(vals, axis=0)
    e = jnp.exp(vv - vv[0][None, :])
    attn_ref[...] = e / jnp.sum(e, axis=0, keepdims=True)
    for kt, gref in enumerate((g0_ref, g1_ref, g2_ref, g3_ref)):
        gref[0] = idxs[kt] + b * M


def _stage2(x2, wq, bq2, knt, pin3, h):
    return pl.pallas_call(
        _stage2_body,
        grid=(B, NBLKH),
        in_specs=[
            pl.BlockSpec((1, C1, TBLK), lambda b, nb: (b, 0, h * NBLKH + nb)),
            pl.BlockSpec((C1, C1), lambda b, nb: (0, 0)),
            pl.BlockSpec((C1, 1), lambda b, nb: (0, 0)),
            pl.BlockSpec((1, C1, M), lambda b, nb: (b, 0, 0)),
            pl.BlockSpec((1, 1, TBLK), lambda b, nb: (h * NBLKH + nb, 0, 0)),
        ],
        out_specs=[
            pl.BlockSpec((1, TBLK), lambda b, nb: (0, b * NBLKH + nb)),
            pl.BlockSpec((1, TBLK), lambda b, nb: (0, b * NBLKH + nb)),
            pl.BlockSpec((1, TBLK), lambda b, nb: (0, b * NBLKH + nb)),
            pl.BlockSpec((1, TBLK), lambda b, nb: (0, b * NBLKH + nb)),
            pl.BlockSpec((KTAP, TBLK), lambda b, nb: (0, b * NBLKH + nb)),
        ],
        out_shape=[
            jax.ShapeDtypeStruct((1, BNH), jnp.int32),
            jax.ShapeDtypeStruct((1, BNH), jnp.int32),
            jax.ShapeDtypeStruct((1, BNH), jnp.int32),
            jax.ShapeDtypeStruct((1, BNH), jnp.int32),
            jax.ShapeDtypeStruct((KTAP, BNH), jnp.float32),
        ],
    )(x2, wq, bq2, knt, pin3)


def _stage3_body(vtab_hbm, g0_hbm, g1_hbm, g2_hbm, g3_hbm, g_hbm,
                 idx_v, rows_v, sems):
    wid = lax.axis_index("s") * NC + lax.axis_index("c")
    gidx_hbms = (g0_hbm, g1_hbm, g2_hbm, g3_hbm)

    for j in range((NUNIT + NW - 1) // NW):
        u = wid + j * NW

        @pl.when(u < NUNIT)
        def _():
            off = u * UCH
            for kt in range(KTAP):
                pltpu.sync_copy(gidx_hbms[kt].at[0, pl.ds(off, UCH)], idx_v[kt])
            HF = UCH // 2
            seq = [(kt, hf) for kt in range(KTAP) for hf in range(2)]

            def gather(i):
                kt, hf = seq[i]
                return pltpu.async_copy(
                    vtab_hbm.at[idx_v[kt].at[pl.ds(hf * HF, HF)]],
                    rows_v[i % 2],
                    sems[i % 2],
                )

            pending = gather(0)
            for i, (kt, hf) in enumerate(seq):
                pending.wait()
                if i + 1 < len(seq):
                    pending = gather(i + 1)
                pltpu.sync_copy(
                    rows_v[i % 2], g_hbm.at[kt, pl.ds(off + hf * HF, HF), :]
                )


@functools.lru_cache(maxsize=1)
def _make_stage3():
    return pl.kernel(
        _stage3_body,
        out_type=jax.ShapeDtypeStruct((KTAP, BNH, CP), jnp.float32),
        mesh=plsc.VectorSubcoreMesh(core_axis_name="c", subcore_axis_name="s"),
        compiler_params=pltpu.CompilerParams(use_tc_tiling_on_sc=True),
        scratch_types=[
            [pltpu.VMEM((UCH,), jnp.int32) for _ in range(KTAP)],
            [pltpu.VMEM((UCH // 2, CP), jnp.float32) for _ in range(2)],
            [pltpu.SemaphoreType.DMA for _ in range(2)],
        ],
    )


def _stage4_body(g_ref, attn_ref, wflat_ref, cb_ref, pfold_ref, pb_ref, out_ref):
    attn = attn_ref[...]
    parts = [g_ref[kt] * attn[kt][:, None] for kt in range(KTAP)]
    weighted = jnp.concatenate(parts, axis=1)
    out1d = (
        jnp.dot(weighted, wflat_ref[...], preferred_element_type=jnp.float32)
        + cb_ref[...]
    )
    out_ref[...] = (
        jnp.dot(out1d, pfold_ref[...], preferred_element_type=jnp.float32)
        + pb_ref[...]
    )


def _stage4(g, attn, wflat, cb2, pfold, pb2):
    nblk2 = BNH // TBLK
    return pl.pallas_call(
        _stage4_body,
        grid=(nblk2,),
        in_specs=[
            pl.BlockSpec((KTAP, TBLK, CP), lambda i: (0, i, 0)),
            pl.BlockSpec((KTAP, TBLK), lambda i: (0, i)),
            pl.BlockSpec((CF, C1), lambda i: (0, 0)),
            pl.BlockSpec((1, C1), lambda i: (0, 0)),
            pl.BlockSpec((C1, CO), lambda i: (0, 0)),
            pl.BlockSpec((1, CO), lambda i: (0, 0)),
        ],
        out_specs=pl.BlockSpec((TBLK, CO), lambda i: (i, 0)),
        out_shape=jax.ShapeDtypeStruct((BNH, CO), jnp.float32),
    )(g, attn, wflat, cb2, pfold, pb2)


def kernel(x, Wq, bq, Wk, bk, Wv, bv, conv_w, conv_b, pw_w, pw_b):
    coords = jnp.broadcast_to(_coords()[None], (B, 2, H, W))
    xc = jnp.concatenate([x, coords], axis=1)
    x2 = (
        xc.reshape(B, C98, H1, SCALE, W1, SCALE)
        .transpose(0, 1, 3, 5, 2, 4)
        .reshape(B, C1, N)
    )
    xs = x2[:, :, jnp.asarray(_FLAT_IDX)]

    wflat = jnp.pad(
        conv_w.transpose(2, 1, 0), ((0, 0), (0, CP - C1), (0, 0))
    ).reshape(CF, C1)
    eye4 = jnp.eye(SCALE * SCALE, dtype=pw_w.dtype)
    pfold = jnp.einsum(
        "oc,st->ctso", pw_w, eye4, precision=jax.lax.Precision.HIGHEST
    ).reshape(C1, CO)
    bq2 = bq.reshape(C1, 1)
    bk2 = bk.reshape(C1, 1)
    bv2 = bv.reshape(C1, 1)
    cb2 = conv_b.reshape(1, C1)
    pb2 = jnp.tile(pw_b, SCALE * SCALE).reshape(1, CO)
    pin3 = jnp.asarray(_PIN).reshape(NBLK, 1, TBLK)

    knt, vflat = _stage1(xs, Wk, bk2, Wv, bv2)

    outs = []
    for h in range(NSPLIT):
        gi0, gi1, gi2, gi3, attn = _stage2(x2, Wq, bq2, knt, pin3, h)
        g = _make_stage3()(vflat, gi0, gi1, gi2, gi3)
        outs.append(_stage4(g, attn, wflat, cb2, pfold, pb2))

    out_tok = jnp.concatenate(
        [o.reshape(B, NH, CO) for o in outs], axis=1
    )

    x5 = (
        out_tok.reshape(B, H1, W1, SCALE, SCALE, C_IN)
        .transpose(0, 5, 1, 3, 2, 4)
        .reshape(B, C_IN, H, W)
    )
    return x5


# speedup vs baseline: 1.0691x; 1.0658x over previous
import functools

import jax
import jax.numpy as jnp
import numpy as np
from jax import lax
from jax.experimental import pallas as pl
from jax.experimental.pallas import tpu as pltpu
from jax.experimental.pallas import tpu_sc as plsc

B = 2
C_IN = 96
H = 224
W = 224
SCALE = 2
KTAP = 4
SAMPLES = 16
C1 = (C_IN + 2) * SCALE * SCALE
H1 = H // SCALE
W1 = W // SCALE
N = H1 * W1
M = SAMPLES * SAMPLES
BN = B * N
CO = 96 * SCALE * SCALE
C98 = C_IN + 2
CP = 512
CF = KTAP * CP

TBLK = 896
NBLK = N // TBLK

NSPLIT = 2
NH = N // NSPLIT
BNH = B * NH
NBLKH = NH // TBLK

NC = 2
NS = 16
NW = NC * NS
PW = BNH // NW
CH = 56
NCHUNK = PW // CH

_x_ind = np.round(np.linspace(0, H1 - 1, SAMPLES)).astype(np.int32)
_y_ind = np.round(np.linspace(0, W1 - 1, SAMPLES)).astype(np.int32)
_xg, _yg = np.meshgrid(_x_ind, _y_ind, indexing="ij")
_FLAT_IDX = (_xg.flatten() * W1 + _yg.flatten()).astype(np.int32)
_PIN = np.full((N,), -1, dtype=np.int32)
_PIN[_FLAT_IDX] = np.arange(M, dtype=np.int32)


def _coords():
    xg, yg = jnp.meshgrid(
        jnp.arange(H, dtype=jnp.float32),
        jnp.arange(W, dtype=jnp.float32),
        indexing="ij",
    )
    xy = jnp.stack([xg, yg], axis=0)
    nrm = jnp.sqrt(jnp.sum(xy * xy, axis=0, keepdims=True))
    return xy / jnp.maximum(nrm, 1e-12)


def _stage1_body(xs_ref, wk_ref, bk_ref, wv_ref, bv_ref, knt_ref, vtab_ref):
    xs = xs_ref[0]
    k = jnp.dot(wk_ref[...], xs, preferred_element_type=jnp.float32) + bk_ref[...]
    knrm = jnp.sqrt(jnp.sum(k * k, axis=0, keepdims=True))
    knt_ref[0] = k / jnp.maximum(knrm, 1e-12)
    v = jnp.dot(wv_ref[...], xs, preferred_element_type=jnp.float32) + bv_ref[...]
    vtab_ref[0] = jnp.pad(v.T, ((0, 0), (0, CP - C1)))


def _stage1(xs, wk, bk2, wv, bv2):
    return pl.pallas_call(
        _stage1_body,
        grid=(B,),
        in_specs=[
            pl.BlockSpec((1, C1, M), lambda b: (b, 0, 0)),
            pl.BlockSpec((C1, C1), lambda b: (0, 0)),
            pl.BlockSpec((C1, 1), lambda b: (0, 0)),
            pl.BlockSpec((C1, C1), lambda b: (0, 0)),
            pl.BlockSpec((C1, 1), lambda b: (0, 0)),
        ],
        out_specs=[
            pl.BlockSpec((1, C1, M), lambda b: (b, 0, 0)),
            pl.BlockSpec((1, M, CP), lambda b: (b, 0, 0)),
        ],
        out_shape=[
            jax.ShapeDtypeStruct((B, C1, M), jnp.float32),
            jax.ShapeDtypeStruct((B, M, CP), jnp.float32),
        ],
    )(xs, wk, bk2, wv, bv2)


def _stage2_body(x_ref, wq_ref, bq_ref, knt_ref, pin_ref, gidx_ref, attn_ref):
    b = pl.program_id(0)
    x = x_ref[0]
    q = jnp.dot(wq_ref[...], x, preferred_element_type=jnp.float32) + bq_ref[...]
    qnrm = jnp.sqrt(jnp.sum(q * q, axis=0, keepdims=True))
    qn = q / jnp.maximum(qnrm, 1e-12)
    sim = lax.dot_general(
        qn, knt_ref[0], (((0,), (0,)), ((), ())),
        preferred_element_type=jnp.float32,
    )
    col = lax.broadcasted_iota(jnp.int32, (TBLK, M), 1)
    pin = pin_ref[0, 0]
    sim = jnp.where(col == pin[:, None], 1e9, sim)
    vals = []
    idxs = []
    for kt in range(KTAP):
        mval = jnp.max(sim, axis=1)
        midx = jnp.min(jnp.where(sim == mval[:, None], col, M), axis=1)
        vals.append(mval)
        idxs.append(midx)
        if kt + 1 < KTAP:
            sim = jnp.where(col == midx[:, None], -jnp.inf, sim)
    vv = jnp.stack(vals, axis=0)
    e = jnp.exp(vv - vv[0][None, :])
    attn_ref[...] = e / jnp.sum(e, axis=0, keepdims=True)
    gidx_ref[...] = jnp.stack(
        [idxs[kt] + b * M for kt in range(KTAP)], axis=0
    )


def _stage2(x2, wq, bq2, knt, pin3, h):
    return pl.pallas_call(
        _stage2_body,
        grid=(B, NBLKH),
        in_specs=[
            pl.BlockSpec((1, C1, TBLK), lambda b, nb: (b, 0, h * NBLKH + nb)),
            pl.BlockSpec((C1, C1), lambda b, nb: (0, 0)),
            pl.BlockSpec((C1, 1), lambda b, nb: (0, 0)),
            pl.BlockSpec((1, C1, M), lambda b, nb: (b, 0, 0)),
            pl.BlockSpec((1, 1, TBLK), lambda b, nb: (h * NBLKH + nb, 0, 0)),
        ],
        out_specs=[
            pl.BlockSpec((KTAP, TBLK), lambda b, nb: (0, b * NBLKH + nb)),
            pl.BlockSpec((KTAP, TBLK), lambda b, nb: (0, b * NBLKH + nb)),
        ],
        out_shape=[
            jax.ShapeDtypeStruct((KTAP, BNH), jnp.int32),
            jax.ShapeDtypeStruct((KTAP, BNH), jnp.float32),
        ],
    )(x2, wq, bq2, knt, pin3)


def _stage3_body(vtab_hbm, gidx_hbm, g_hbm, idx_v, rows_v, sem):
    wid = lax.axis_index("s") * NC + lax.axis_index("c")
    base = wid * PW

    def chunk(i, _):
        off = base + i * CH
        for kt in range(KTAP):
            pltpu.sync_copy(gidx_hbm.at[pl.ds(kt * BNH + off, CH)], idx_v[kt])
        copies = [
            pltpu.async_copy(vtab_hbm.at[idx_v[kt]], rows_v[kt], sem)
            for kt in range(KTAP)
        ]
        for c in copies:
            c.wait()
        for kt in range(KTAP):
            pltpu.sync_copy(rows_v[kt], g_hbm.at[kt, pl.ds(off, CH), :])
        return 0

    lax.fori_loop(0, NCHUNK, chunk, 0)


@functools.lru_cache(maxsize=1)
def _make_stage3():
    return pl.kernel(
        _stage3_body,
        out_type=jax.ShapeDtypeStruct((KTAP, BNH, CP), jnp.float32),
        mesh=plsc.VectorSubcoreMesh(core_axis_name="c", subcore_axis_name="s"),
        compiler_params=pltpu.CompilerParams(use_tc_tiling_on_sc=True),
        scratch_types=[
            [pltpu.VMEM((CH,), jnp.int32) for _ in range(KTAP)],
            [pltpu.VMEM((CH, CP), jnp.float32) for _ in range(KTAP)],
            pltpu.SemaphoreType.DMA,
        ],
    )


def _stage4_body(g_ref, attn_ref, wflat_ref, cb_ref, pfold_ref, pb_ref, out_ref):
    attn = attn_ref[...]
    parts = [g_ref[kt] * attn[kt][:, None] for kt in range(KTAP)]
    weighted = jnp.concatenate(parts, axis=1)
    out1d = (
        jnp.dot(weighted, wflat_ref[...], preferred_element_type=jnp.float32)
        + cb_ref[...]
    )
    out_ref[...] = (
        jnp.dot(out1d, pfold_ref[...], preferred_element_type=jnp.float32)
        + pb_ref[...]
    )


def _stage4(g, attn, wflat, cb2, pfold, pb2):
    nblk2 = BNH // TBLK
    return pl.pallas_call(
        _stage4_body,
        grid=(nblk2,),
        in_specs=[
            pl.BlockSpec((KTAP, TBLK, CP), lambda i: (0, i, 0)),
            pl.BlockSpec((KTAP, TBLK), lambda i: (0, i)),
            pl.BlockSpec((CF, C1), lambda i: (0, 0)),
            pl.BlockSpec((1, C1), lambda i: (0, 0)),
            pl.BlockSpec((C1, CO), lambda i: (0, 0)),
            pl.BlockSpec((1, CO), lambda i: (0, 0)),
        ],
        out_specs=pl.BlockSpec((TBLK, CO), lambda i: (i, 0)),
        out_shape=jax.ShapeDtypeStruct((BNH, CO), jnp.float32),
    )(g, attn, wflat, cb2, pfold, pb2)


def kernel(x, Wq, bq, Wk, bk, Wv, bv, conv_w, conv_b, pw_w, pw_b):
    coords = jnp.broadcast_to(_coords()[None], (B, 2, H, W))
    xc = jnp.concatenate([x, coords], axis=1)
    x2 = (
        xc.reshape(B, C98, H1, SCALE, W1, SCALE)
        .transpose(0, 1, 3, 5, 2, 4)
        .reshape(B, C1, N)
    )
    xs = x2[:, :, jnp.asarray(_FLAT_IDX)]

    wflat = jnp.pad(
        conv_w.transpose(2, 1, 0), ((0, 0), (0, CP - C1), (0, 0))
    ).reshape(CF, C1)
    eye4 = jnp.eye(SCALE * SCALE, dtype=pw_w.dtype)
    pfold = jnp.einsum(
        "oc,st->ctso", pw_w, eye4, precision=jax.lax.Precision.HIGHEST
    ).reshape(C1, CO)
    bq2 = bq.reshape(C1, 1)
    bk2 = bk.reshape(C1, 1)
    bv2 = bv.reshape(C1, 1)
    cb2 = conv_b.reshape(1, C1)
    pb2 = jnp.tile(pw_b, SCALE * SCALE).reshape(1, CO)
    pin3 = jnp.asarray(_PIN).reshape(NBLK, 1, TBLK)

    knt, vtab = _stage1(xs, Wk, bk2, Wv, bv2)
    vflat = vtab.reshape(B * M, CP)

    outs = []
    for h in range(NSPLIT):
        gidx, attn = _stage2(x2, Wq, bq2, knt, pin3, h)
        g = _make_stage3()(vflat, gidx.reshape(KTAP * BNH))
        outs.append(_stage4(g, attn, wflat, cb2, pfold, pb2))

    out_tok = jnp.concatenate(
        [o.reshape(B, NH, CO) for o in outs], axis=1
    )

    x5 = (
        out_tok.reshape(B, H1, W1, SCALE, SCALE, C_IN)
        .transpose(0, 5, 1, 3, 2, 4)
        .reshape(B, C_IN, H, W)
    )
    return x5
